# baseline (speedup 1.0000x reference)
import functools

import jax
import jax.numpy as jnp
from jax import lax
from jax.experimental import pallas as pl
from jax.experimental.pallas import tpu as pltpu, tpu_sc as plsc

NUM_NODES = 10000
D_FEAT = 128
NUM_EDGES = 320000

_NC = 2
_NS = 16
_NW = _NC * _NS
_B_PER_W = NUM_EDGES // _NW
_C = 16
_N_CHUNKS = _B_PER_W // _C
_NBUF = 5
_G = _N_CHUNKS // _NBUF
_LAG = 4


def _make_gather2():
    mesh = plsc.VectorSubcoreMesh(core_axis_name="c", subcore_axis_name="s")

    scratch = [
        pltpu.VMEM((_B_PER_W,), jnp.int32),
        pltpu.VMEM((_B_PER_W,), jnp.int32),
        pltpu.VMEM((_NBUF, _C, D_FEAT), jnp.float32),
        pltpu.VMEM((_NBUF, _C, D_FEAT), jnp.float32),
        pltpu.VMEM_SHARED((NUM_NODES, D_FEAT), jnp.float32),
    ] + [pltpu.SemaphoreType.DMA] * (4 * _NBUF + 2)

    @functools.partial(
        pl.kernel,
        mesh=mesh,
        out_type=[
            jax.ShapeDtypeStruct((NUM_EDGES, D_FEAT), jnp.float32),
            jax.ShapeDtypeStruct((NUM_EDGES, D_FEAT), jnp.float32),
        ],
        scratch_types=scratch,
    )
    def gather2(src_hbm, dst_hbm, table_hbm, out_i, out_j, *scr):
        idx_s, idx_d, rows_s, rows_d, tbl_sh = scr[0:5]
        sem_gs = scr[5:5 + _NBUF]
        sem_gd = scr[5 + _NBUF:5 + 2 * _NBUF]
        sem_ws = scr[5 + 2 * _NBUF:5 + 3 * _NBUF]
        sem_wd = scr[5 + 3 * _NBUF:5 + 4 * _NBUF]
        sem_stage, sem_idx = scr[5 + 4 * _NBUF:5 + 4 * _NBUF + 2]

        wid = lax.axis_index("s") * _NC + lax.axis_index("c")
        base = wid * _B_PER_W

        sid = lax.axis_index("s")
        rows_per_tile = 624
        stg = pl.ds(sid * rows_per_tile, rows_per_tile)
        cp_stage = pltpu.async_copy(table_hbm.at[stg], tbl_sh.at[stg], sem_stage)
        cp_is = pltpu.async_copy(src_hbm.at[wid], idx_s, sem_idx)
        cp_id = pltpu.async_copy(dst_hbm.at[wid], idx_d, sem_idx)

        @pl.when(sid == 0)
        def _copy_tail():
            tail = pl.ds(_NS * rows_per_tile, NUM_NODES - _NS * rows_per_tile)
            pltpu.async_copy(table_hbm.at[tail], tbl_sh.at[tail], sem_stage).wait()

        cp_stage.wait()
        cp_is.wait()
        cp_id.wait()
        plsc.subcore_barrier()

        def start_gather(ch, b):
            sl = pl.ds(ch * _C, _C)
            pltpu.async_copy(tbl_sh.at[idx_s.at[sl]], rows_s.at[b], sem_gs[b])
            pltpu.async_copy(tbl_sh.at[idx_d.at[sl]], rows_d.at[b], sem_gd[b])

        def wait_gather(b):
            sl = pl.ds(0, _C)
            pltpu.make_async_copy(tbl_sh.at[idx_s.at[sl]], rows_s.at[b], sem_gs[b]).wait()
            pltpu.make_async_copy(tbl_sh.at[idx_d.at[sl]], rows_d.at[b], sem_gd[b]).wait()

        def start_writeout(ch, b):
            off = base + ch * _C
            pltpu.async_copy(rows_s.at[b], out_j.at[pl.ds(off, _C)], sem_ws[b])
            pltpu.async_copy(rows_d.at[b], out_i.at[pl.ds(off, _C)], sem_wd[b])

        def wait_writeout(b):
            pltpu.make_async_copy(rows_s.at[b], out_j.at[pl.ds(0, _C)], sem_ws[b]).wait()
            pltpu.make_async_copy(rows_d.at[b], out_i.at[pl.ds(0, _C)], sem_wd[b]).wait()

        for b in range(_NBUF):
            start_gather(b, b)

        for b in range(_NBUF):
            if b >= _LAG:
                wait_writeout((b - _LAG) % _NBUF)
                start_gather(b - _LAG + _NBUF, (b - _LAG) % _NBUF)
            wait_gather(b)
            start_writeout(b, b)

        def body(g, carry):
            ch0 = g * _NBUF
            for b in range(_NBUF):
                ch = ch0 + b
                pb = (b + _NBUF - _LAG) % _NBUF
                wait_writeout(pb)
                start_gather(ch - _LAG + _NBUF, pb)
                wait_gather(b)
                start_writeout(ch, b)
            return carry

        lax.fori_loop(1, _G - 1, body, 0)

        ch0 = (_G - 1) * _NBUF
        for b in range(_NBUF):
            ch = ch0 + b
            pb = (b + _NBUF - _LAG) % _NBUF
            wait_writeout(pb)
            if ch - _LAG + _NBUF <= _N_CHUNKS - 1:
                start_gather(ch - _LAG + _NBUF, pb)
            wait_gather(b)
            start_writeout(ch, b)

        for ch in range(_N_CHUNKS - _LAG, _N_CHUNKS):
            wait_writeout(ch % _NBUF)

    return gather2


_gather2 = _make_gather2()


def kernel(Wx, edge_index):
    idx = edge_index.astype(jnp.int32).reshape(2, _NW, _B_PER_W)
    out_i, out_j = _gather2(idx[0], idx[1], Wx)
    return (out_i, out_j)


# speedup vs baseline: 1.0159x; 1.0159x over previous
import functools

import jax
import jax.numpy as jnp
from jax import lax
from jax.experimental import pallas as pl
from jax.experimental.pallas import tpu as pltpu, tpu_sc as plsc

NUM_NODES = 10000
D_FEAT = 128
NUM_EDGES = 320000

_NC = 2
_NS = 16
_NW = _NC * _NS
_B_PER_W = NUM_EDGES // _NW
_C = 16
_N_CHUNKS = _B_PER_W // _C
_NBUF = 5
_G = _N_CHUNKS // _NBUF
_LAG = 3


def _make_gather2():
    mesh = plsc.VectorSubcoreMesh(core_axis_name="c", subcore_axis_name="s")

    scratch = [
        pltpu.VMEM((_B_PER_W,), jnp.int32),
        pltpu.VMEM((_B_PER_W,), jnp.int32),
        pltpu.VMEM((_NBUF, _C, D_FEAT), jnp.float32),
        pltpu.VMEM((_NBUF, _C, D_FEAT), jnp.float32),
        pltpu.VMEM_SHARED((NUM_NODES, D_FEAT), jnp.float32),
    ] + [pltpu.SemaphoreType.DMA] * (4 * _NBUF + 2)

    @functools.partial(
        pl.kernel,
        mesh=mesh,
        out_type=[
            jax.ShapeDtypeStruct((NUM_EDGES, D_FEAT), jnp.float32),
            jax.ShapeDtypeStruct((NUM_EDGES, D_FEAT), jnp.float32),
        ],
        scratch_types=scratch,
    )
    def gather2(src_hbm, dst_hbm, table_hbm, out_i, out_j, *scr):
        idx_s, idx_d, rows_s, rows_d, tbl_sh = scr[0:5]
        sem_gs = scr[5:5 + _NBUF]
        sem_gd = scr[5 + _NBUF:5 + 2 * _NBUF]
        sem_ws = scr[5 + 2 * _NBUF:5 + 3 * _NBUF]
        sem_wd = scr[5 + 3 * _NBUF:5 + 4 * _NBUF]
        sem_stage, sem_idx = scr[5 + 4 * _NBUF:5 + 4 * _NBUF + 2]

        wid = lax.axis_index("s") * _NC + lax.axis_index("c")
        base = wid * _B_PER_W

        sid = lax.axis_index("s")
        rows_per_tile = 624
        stg = pl.ds(sid * rows_per_tile, rows_per_tile)
        cp_stage = pltpu.async_copy(table_hbm.at[stg], tbl_sh.at[stg], sem_stage)
        cp_is = pltpu.async_copy(src_hbm.at[wid], idx_s, sem_idx)
        cp_id = pltpu.async_copy(dst_hbm.at[wid], idx_d, sem_idx)

        @pl.when(sid == 0)
        def _copy_tail():
            tail = pl.ds(_NS * rows_per_tile, NUM_NODES - _NS * rows_per_tile)
            pltpu.async_copy(table_hbm.at[tail], tbl_sh.at[tail], sem_stage).wait()

        cp_stage.wait()
        cp_is.wait()
        cp_id.wait()
        plsc.subcore_barrier()

        def start_gather(ch, b):
            sl = pl.ds(ch * _C, _C)
            pltpu.async_copy(tbl_sh.at[idx_s.at[sl]], rows_s.at[b], sem_gs[b])
            pltpu.async_copy(tbl_sh.at[idx_d.at[sl]], rows_d.at[b], sem_gd[b])

        def wait_gather(b):
            sl = pl.ds(0, _C)
            pltpu.make_async_copy(tbl_sh.at[idx_s.at[sl]], rows_s.at[b], sem_gs[b]).wait()
            pltpu.make_async_copy(tbl_sh.at[idx_d.at[sl]], rows_d.at[b], sem_gd[b]).wait()

        def start_writeout(ch, b):
            off = base + ch * _C
            pltpu.async_copy(rows_s.at[b], out_j.at[pl.ds(off, _C)], sem_ws[b])
            pltpu.async_copy(rows_d.at[b], out_i.at[pl.ds(off, _C)], sem_wd[b])

        def wait_writeout(b):
            pltpu.make_async_copy(rows_s.at[b], out_j.at[pl.ds(0, _C)], sem_ws[b]).wait()
            pltpu.make_async_copy(rows_d.at[b], out_i.at[pl.ds(0, _C)], sem_wd[b]).wait()

        for b in range(_NBUF):
            start_gather(b, b)

        for b in range(_NBUF):
            if b >= _LAG:
                wait_writeout((b - _LAG) % _NBUF)
                start_gather(b - _LAG + _NBUF, (b - _LAG) % _NBUF)
            wait_gather(b)
            start_writeout(b, b)

        def body(g, carry):
            ch0 = g * _NBUF
            for b in range(_NBUF):
                ch = ch0 + b
                pb = (b + _NBUF - _LAG) % _NBUF
                wait_writeout(pb)
                start_gather(ch - _LAG + _NBUF, pb)
                wait_gather(b)
                start_writeout(ch, b)
            return carry

        lax.fori_loop(1, _G - 1, body, 0)

        ch0 = (_G - 1) * _NBUF
        for b in range(_NBUF):
            ch = ch0 + b
            pb = (b + _NBUF - _LAG) % _NBUF
            wait_writeout(pb)
            if ch - _LAG + _NBUF <= _N_CHUNKS - 1:
                start_gather(ch - _LAG + _NBUF, pb)
            wait_gather(b)
            start_writeout(ch, b)

        for ch in range(_N_CHUNKS - _LAG, _N_CHUNKS):
            wait_writeout(ch % _NBUF)

    return gather2


_gather2 = _make_gather2()


def kernel(Wx, edge_index):
    idx = edge_index.astype(jnp.int32).reshape(2, _NW, _B_PER_W)
    out_i, out_j = _gather2(idx[0], idx[1], Wx)
    return (out_i, out_j)
